# Initial kernel scaffold; baseline (speedup 1.0000x reference)
import functools

import jax
import jax.numpy as jnp
from jax import lax
from jax.experimental import pallas as pl
from jax.experimental.pallas import tpu as pltpu
from jax.experimental.pallas import tpu_sc as plsc

N = 10000
E = 160000
C_IN = 256
CB = 64

NC = 2
NS = 16
NW = NC * NS
CHUNK = 128
K = 40
EP = NW * K * CHUNK
NPAD = 10016
ROWS_PER_TILE = NPAD // NS
JUNK_ROW = 10008
DEGW = 16
ZR = 64


def _seg_body(with_deg, table, srcm, dstm, *rest):
    if with_deg:
        (agg_out, deg_out, idx_s, idx_d, rows, zbuf, ones, zbufd,
         acc, dacc, sem) = rest
    else:
        agg_out, idx_s, idx_d, rows, zbuf, acc, sem = rest
    c = lax.axis_index("c")
    s = lax.axis_index("s")
    wid = c * NS + s

    zero16 = jnp.zeros((16,), jnp.float32)
    for r in range(ZR):
        for q in range(CB // 16):
            zbuf[r, pl.ds(q * 16, 16)] = zero16
    if with_deg:
        one16 = jnp.full((16,), 1.0, jnp.float32)
        for r in range(CHUNK):
            ones[r, pl.ds(0, 16)] = one16
        for r in range(ZR):
            zbufd[r, pl.ds(0, 16)] = zero16

    row0 = s * ROWS_PER_TILE
    nfull = ROWS_PER_TILE // ZR
    rem = ROWS_PER_TILE - nfull * ZR
    for k in range(nfull):
        pltpu.sync_copy(zbuf, acc.at[pl.ds(row0 + k * ZR, ZR)])
    pltpu.sync_copy(zbuf.at[pl.ds(0, rem)],
                    acc.at[pl.ds(row0 + nfull * ZR, rem)])
    if with_deg:
        for k in range(nfull):
            pltpu.sync_copy(zbufd, dacc.at[pl.ds(row0 + k * ZR, ZR)])
        pltpu.sync_copy(zbufd.at[pl.ds(0, rem)],
                        dacc.at[pl.ds(row0 + nfull * ZR, rem)])
    plsc.subcore_barrier()

    pltpu.sync_copy(srcm.at[pl.ds(wid * K, K)], idx_s)
    pltpu.sync_copy(dstm.at[pl.ds(wid * K, K)], idx_d)

    def step(j, carry):
        pltpu.async_copy(table.at[idx_s.at[j]], rows, sem).wait()
        pltpu.sync_copy(rows, acc.at[idx_d.at[j]], add=True)
        if with_deg:
            pltpu.sync_copy(ones, dacc.at[idx_d.at[j]], add=True)
        return carry

    lax.fori_loop(0, K, step, 0)
    plsc.subcore_barrier()

    pltpu.sync_copy(acc.at[pl.ds(row0, ROWS_PER_TILE)],
                    agg_out.at[c, pl.ds(row0, ROWS_PER_TILE)])
    if with_deg:
        pltpu.sync_copy(dacc.at[pl.ds(row0, ROWS_PER_TILE)],
                        deg_out.at[c, pl.ds(row0, ROWS_PER_TILE)])


def _make_seg_kernel(with_deg):
    mesh = plsc.VectorSubcoreMesh(core_axis_name="c", subcore_axis_name="s")
    out_type = [jax.ShapeDtypeStruct((NC, NPAD, CB), jnp.float32)]
    scratch = [
        pltpu.VMEM((K, CHUNK), jnp.int32),
        pltpu.VMEM((K, CHUNK), jnp.int32),
        pltpu.VMEM((CHUNK, CB), jnp.float32),
        pltpu.VMEM((ZR, CB), jnp.float32),
    ]
    if with_deg:
        out_type.append(jax.ShapeDtypeStruct((NC, NPAD, DEGW), jnp.float32))
        scratch += [
            pltpu.VMEM((CHUNK, DEGW), jnp.float32),
            pltpu.VMEM((ZR, DEGW), jnp.float32),
        ]
    scratch.append(pltpu.VMEM_SHARED((NPAD, CB), jnp.float32))
    if with_deg:
        scratch.append(pltpu.VMEM_SHARED((NPAD, DEGW), jnp.float32))
    scratch.append(pltpu.SemaphoreType.DMA)
    return pl.kernel(
        functools.partial(_seg_body, with_deg),
        out_type=out_type,
        mesh=mesh,
        scratch_types=scratch,
    )


_seg_deg = _make_seg_kernel(True)
_seg = _make_seg_kernel(False)


def _mm_body(x_ref, ws_ref, wn_ref, s_ref, y_ref):
    x = x_ref[...]
    s_ref[...] = jnp.dot(x, ws_ref[...], preferred_element_type=jnp.float32)
    y_ref[...] = jnp.dot(x, wn_ref[...], preferred_element_type=jnp.float32)


def _bn1_body(s1_ref, aggp_ref, degp_ref, ba_ref, ga_ref, be_ref, h_ref):
    agg = aggp_ref[0, :N, :] + aggp_ref[1, :N, :]
    deg = degp_ref[0, :N, 0:1] + degp_ref[1, :N, 0:1]
    dinv = 1.0 / jnp.clip(deg, 1.0, None)
    conv = s1_ref[...] + agg * dinv + ba_ref[...]
    mean = jnp.mean(conv, axis=0, keepdims=True)
    var = jnp.var(conv, axis=0, keepdims=True)
    bn = (conv - mean) * lax.rsqrt(var + 1e-5) * ga_ref[...] + be_ref[...]
    h_ref[...] = jnp.maximum(bn, 0.0)


def _bn2_body(h_ref, aggp_ref, degp_ref, data_ref, wbs_ref, wbn_ref,
              bb_ref, gb_ref, be_ref, out_ref):
    agg = aggp_ref[0, :N, :] + aggp_ref[1, :N, :]
    deg = degp_ref[0, :N, 0:1] + degp_ref[1, :N, 0:1]
    dinv = 1.0 / jnp.clip(deg, 1.0, None)
    conv = (jnp.dot(h_ref[...], wbs_ref[...],
                    preferred_element_type=jnp.float32)
            + jnp.dot(agg * dinv, wbn_ref[...],
                      preferred_element_type=jnp.float32)
            + bb_ref[...])
    mean = jnp.mean(conv, axis=0, keepdims=True)
    var = jnp.var(conv, axis=0, keepdims=True)
    bn = (conv - mean) * lax.rsqrt(var + 1e-5) * gb_ref[...] + be_ref[...]
    out_ref[...] = jnp.maximum(bn + data_ref[...], 0.0)


def kernel(data, edge_index, depth, Wa_self, Wa_neigh, ba, gamma_a, beta_a,
           Wb_self, Wb_neigh, bb, gamma_b, beta_b):
    del depth
    src = edge_index[0].astype(jnp.int32)
    dst = edge_index[1].astype(jnp.int32)
    pad = EP - E
    srcm = jnp.concatenate([src, jnp.zeros((pad,), jnp.int32)]).reshape(
        NW * K, CHUNK)
    dstm = jnp.concatenate([dst, jnp.full((pad,), JUNK_ROW, jnp.int32)]
                           ).reshape(NW * K, CHUNK)

    s1, y1 = pl.pallas_call(
        _mm_body,
        grid=(25,),
        in_specs=[
            pl.BlockSpec((400, C_IN), lambda i: (i, 0)),
            pl.BlockSpec((C_IN, CB), lambda i: (0, 0)),
            pl.BlockSpec((C_IN, CB), lambda i: (0, 0)),
        ],
        out_specs=[
            pl.BlockSpec((400, CB), lambda i: (i, 0)),
            pl.BlockSpec((400, CB), lambda i: (i, 0)),
        ],
        out_shape=[jax.ShapeDtypeStruct((N, CB), jnp.float32)] * 2,
    )(data, Wa_self, Wa_neigh)

    agg1_p, deg_p = _seg_deg(y1, srcm, dstm)

    h = pl.pallas_call(
        _bn1_body,
        out_shape=jax.ShapeDtypeStruct((N, CB), jnp.float32),
    )(s1, agg1_p, deg_p, ba.reshape(1, CB), gamma_a.reshape(1, CB),
      beta_a.reshape(1, CB))

    agg2_p = _seg(h, srcm, dstm)

    out = pl.pallas_call(
        _bn2_body,
        out_shape=jax.ShapeDtypeStruct((N, C_IN), jnp.float32),
    )(h, agg2_p, deg_p, data, Wb_self, Wb_neigh, bb.reshape(1, C_IN),
      gamma_b.reshape(1, C_IN), beta_b.reshape(1, C_IN))
    return out


# speedup vs baseline: 4.9414x; 4.9414x over previous
import functools

import jax
import jax.numpy as jnp
from jax import lax
from jax.experimental import pallas as pl
from jax.experimental.pallas import tpu as pltpu
from jax.experimental.pallas import tpu_sc as plsc

N = 10000
E = 160000
C_IN = 256
CB = 64

NC = 2
NS = 16
NW = NC * NS
CHUNK = 128
K = 40
EP = NW * K * CHUNK
NPAD = 10112
ROWS_PER_TILE = NPAD // NS
JUNK_ROW = 10008
DEGW = 16
ZR = 64


def _seg_body(with_deg, table, srcm, dstm, *rest):
    if with_deg:
        (agg_out, deg_out, idx_s, idx_d, rows, zbuf, ones, zbufd,
         acc, dacc, sem) = rest
    else:
        agg_out, idx_s, idx_d, rows, zbuf, acc, sem = rest
    c = lax.axis_index("c")
    s = lax.axis_index("s")
    wid = c * NS + s

    zero16 = jnp.zeros((16,), jnp.float32)
    for r in range(ZR):
        for q in range(CB // 16):
            zbuf[r, pl.ds(q * 16, 16)] = zero16
    if with_deg:
        one16 = jnp.full((16,), 1.0, jnp.float32)
        for r in range(CHUNK):
            ones[r, pl.ds(0, 16)] = one16
        for r in range(ZR):
            zbufd[r, pl.ds(0, 16)] = zero16

    row0 = s * ROWS_PER_TILE
    nfull = ROWS_PER_TILE // ZR
    rem = ROWS_PER_TILE - nfull * ZR
    for k in range(nfull):
        pltpu.sync_copy(zbuf, acc.at[pl.ds(row0 + k * ZR, ZR)])
    pltpu.sync_copy(zbuf.at[pl.ds(0, rem)],
                    acc.at[pl.ds(row0 + nfull * ZR, rem)])
    if with_deg:
        for k in range(nfull):
            pltpu.sync_copy(zbufd, dacc.at[pl.ds(row0 + k * ZR, ZR)])
        pltpu.sync_copy(zbufd.at[pl.ds(0, rem)],
                        dacc.at[pl.ds(row0 + nfull * ZR, rem)])
    plsc.subcore_barrier()

    pltpu.sync_copy(srcm.at[pl.ds(wid * K, K)], idx_s)
    pltpu.sync_copy(dstm.at[pl.ds(wid * K, K)], idx_d)

    def step(j, carry):
        pltpu.async_copy(table.at[idx_s.at[j]], rows, sem).wait()
        pltpu.sync_copy(rows, acc.at[idx_d.at[j]], add=True)
        if with_deg:
            pltpu.sync_copy(ones, dacc.at[idx_d.at[j]], add=True)
        return carry

    lax.fori_loop(0, K, step, 0)
    plsc.subcore_barrier()

    pltpu.sync_copy(acc.at[pl.ds(row0, ROWS_PER_TILE)],
                    agg_out.at[c, pl.ds(row0, ROWS_PER_TILE)])
    if with_deg:
        pltpu.sync_copy(dacc.at[pl.ds(row0, ROWS_PER_TILE)],
                        deg_out.at[c, pl.ds(row0, ROWS_PER_TILE)])


def _make_seg_kernel(with_deg):
    mesh = plsc.VectorSubcoreMesh(core_axis_name="c", subcore_axis_name="s")
    out_type = [jax.ShapeDtypeStruct((NC, NPAD, CB), jnp.float32)]
    scratch = [
        pltpu.VMEM((K, CHUNK), jnp.int32),
        pltpu.VMEM((K, CHUNK), jnp.int32),
        pltpu.VMEM((CHUNK, CB), jnp.float32),
        pltpu.VMEM((ZR, CB), jnp.float32),
    ]
    if with_deg:
        out_type.append(jax.ShapeDtypeStruct((NC, NPAD, DEGW), jnp.float32))
        scratch += [
            pltpu.VMEM((CHUNK, DEGW), jnp.float32),
            pltpu.VMEM((ZR, DEGW), jnp.float32),
        ]
    scratch.append(pltpu.VMEM_SHARED((NPAD, CB), jnp.float32))
    if with_deg:
        scratch.append(pltpu.VMEM_SHARED((NPAD, DEGW), jnp.float32))
    scratch.append(pltpu.SemaphoreType.DMA)
    return pl.kernel(
        functools.partial(_seg_body, with_deg),
        out_type=out_type,
        mesh=mesh,
        scratch_types=scratch,
        compiler_params=pltpu.CompilerParams(use_tc_tiling_on_sc=False),
    )


_seg_deg = _make_seg_kernel(True)
_seg = _make_seg_kernel(False)


def _mm_body(x_ref, ws_ref, wn_ref, s_ref, y_ref):
    x = x_ref[...]
    s_ref[...] = jnp.dot(x, ws_ref[...], preferred_element_type=jnp.float32)
    y_ref[...] = jnp.dot(x, wn_ref[...], preferred_element_type=jnp.float32)


def _bn1_body(s1_ref, aggp_ref, degp_ref, ba_ref, ga_ref, be_ref, h_ref):
    agg = aggp_ref[0, :N, :] + aggp_ref[1, :N, :]
    deg = degp_ref[0, :N, 0:1] + degp_ref[1, :N, 0:1]
    dinv = 1.0 / jnp.clip(deg, 1.0, None)
    conv = s1_ref[...] + agg * dinv + ba_ref[...]
    mean = jnp.mean(conv, axis=0, keepdims=True)
    var = jnp.var(conv, axis=0, keepdims=True)
    bn = (conv - mean) * lax.rsqrt(var + 1e-5) * ga_ref[...] + be_ref[...]
    h_ref[...] = jnp.maximum(bn, 0.0)


def _bn2_body(h_ref, aggp_ref, degp_ref, data_ref, wbs_ref, wbn_ref,
              bb_ref, gb_ref, be_ref, out_ref):
    agg = aggp_ref[0, :N, :] + aggp_ref[1, :N, :]
    deg = degp_ref[0, :N, 0:1] + degp_ref[1, :N, 0:1]
    dinv = 1.0 / jnp.clip(deg, 1.0, None)
    conv = (jnp.dot(h_ref[...], wbs_ref[...],
                    preferred_element_type=jnp.float32)
            + jnp.dot(agg * dinv, wbn_ref[...],
                      preferred_element_type=jnp.float32)
            + bb_ref[...])
    mean = jnp.mean(conv, axis=0, keepdims=True)
    var = jnp.var(conv, axis=0, keepdims=True)
    bn = (conv - mean) * lax.rsqrt(var + 1e-5) * gb_ref[...] + be_ref[...]
    out_ref[...] = jnp.maximum(bn + data_ref[...], 0.0)


def kernel(data, edge_index, depth, Wa_self, Wa_neigh, ba, gamma_a, beta_a,
           Wb_self, Wb_neigh, bb, gamma_b, beta_b):
    del depth
    src = edge_index[0].astype(jnp.int32)
    dst = edge_index[1].astype(jnp.int32)
    pad = EP - E
    srcm = jnp.concatenate([src, jnp.zeros((pad,), jnp.int32)]).reshape(
        NW * K, CHUNK)
    dstm = jnp.concatenate([dst, jnp.full((pad,), JUNK_ROW, jnp.int32)]
                           ).reshape(NW * K, CHUNK)

    s1, y1 = pl.pallas_call(
        _mm_body,
        grid=(25,),
        in_specs=[
            pl.BlockSpec((400, C_IN), lambda i: (i, 0)),
            pl.BlockSpec((C_IN, CB), lambda i: (0, 0)),
            pl.BlockSpec((C_IN, CB), lambda i: (0, 0)),
        ],
        out_specs=[
            pl.BlockSpec((400, CB), lambda i: (i, 0)),
            pl.BlockSpec((400, CB), lambda i: (i, 0)),
        ],
        out_shape=[jax.ShapeDtypeStruct((N, CB), jnp.float32)] * 2,
    )(data, Wa_self, Wa_neigh)

    agg1_p, deg_p = _seg_deg(y1, srcm, dstm)

    h = pl.pallas_call(
        _bn1_body,
        out_shape=jax.ShapeDtypeStruct((N, CB), jnp.float32),
    )(s1, agg1_p, deg_p, ba.reshape(1, CB), gamma_a.reshape(1, CB),
      beta_a.reshape(1, CB))

    (agg2_p,) = _seg(h, srcm, dstm)

    out = pl.pallas_call(
        _bn2_body,
        out_shape=jax.ShapeDtypeStruct((N, C_IN), jnp.float32),
    )(h, agg2_p, deg_p, data, Wb_self, Wb_neigh, bb.reshape(1, C_IN),
      gamma_b.reshape(1, C_IN), beta_b.reshape(1, C_IN))
    return out


# speedup vs baseline: 5.5638x; 1.1260x over previous
import functools

import jax
import jax.numpy as jnp
from jax import lax
from jax.experimental import pallas as pl
from jax.experimental.pallas import tpu as pltpu
from jax.experimental.pallas import tpu_sc as plsc

N = 10000
E = 160000
C_IN = 256
CB = 64

NC = 2
NS = 16
NW = NC * NS
CHUNK = 128
K = 40
EP = NW * K * CHUNK
NPAD = 10112
ROWS_PER_TILE = NPAD // NS
JUNK_ROW = 10008
DEGW = 16
ZR = 64


NBUF = 4


def _seg_body(with_deg, table, srcm, dstm, *rest):
    if with_deg:
        (agg_out, deg_out, idx_s, idx_d, r0, r1, r2, r3, zbuf, ones, zbufd,
         acc, dacc, gsem, ssem, dsem) = rest
    else:
        (agg_out, idx_s, idx_d, r0, r1, r2, r3, zbuf,
         acc, gsem, ssem, dsem) = rest
    rows = (r0, r1, r2, r3)
    c = lax.axis_index("c")
    s = lax.axis_index("s")
    wid = c * NS + s

    zero16 = jnp.zeros((16,), jnp.float32)
    for r in range(ZR):
        for q in range(CB // 16):
            zbuf[r, pl.ds(q * 16, 16)] = zero16
    if with_deg:
        one16 = jnp.full((16,), 1.0, jnp.float32)
        for r in range(CHUNK):
            ones[r, pl.ds(0, 16)] = one16
        for r in range(ZR):
            zbufd[r, pl.ds(0, 16)] = zero16

    row0 = s * ROWS_PER_TILE
    nfull = ROWS_PER_TILE // ZR
    rem = ROWS_PER_TILE - nfull * ZR
    for k in range(nfull):
        pltpu.sync_copy(zbuf, acc.at[pl.ds(row0 + k * ZR, ZR)])
    pltpu.sync_copy(zbuf.at[pl.ds(0, rem)],
                    acc.at[pl.ds(row0 + nfull * ZR, rem)])
    if with_deg:
        for k in range(nfull):
            pltpu.sync_copy(zbufd, dacc.at[pl.ds(row0 + k * ZR, ZR)])
        pltpu.sync_copy(zbufd.at[pl.ds(0, rem)],
                        dacc.at[pl.ds(row0 + nfull * ZR, rem)])
    plsc.subcore_barrier()

    pltpu.sync_copy(srcm.at[pl.ds(wid * K, K)], idx_s)
    pltpu.sync_copy(dstm.at[pl.ds(wid * K, K)], idx_d)

    for b in range(NBUF):
        pltpu.async_copy(table.at[idx_s.at[b]], rows[b], gsem)

    def block(m, carry):
        j0 = m * NBUF
        for b in range(NBUF):
            j = j0 + b
            pltpu.make_async_copy(table.at[idx_s.at[j]], rows[b],
                                  gsem).wait()
            pltpu.async_copy(rows[b], acc.at[idx_d.at[j]], ssem, add=True)
            if with_deg:
                pltpu.async_copy(ones, dacc.at[idx_d.at[j]], dsem, add=True)
        for b in range(NBUF):
            j = j0 + b
            pltpu.make_async_copy(rows[b], acc.at[idx_d.at[j]], ssem).wait()

            @pl.when(j + NBUF < K)
            def _():
                pltpu.async_copy(table.at[idx_s.at[j + NBUF]], rows[b], gsem)
        return carry

    lax.fori_loop(0, K // NBUF, block, 0)
    if with_deg:
        def drain(j, carry):
            pltpu.make_async_copy(ones, dacc.at[idx_d.at[0]], dsem).wait()
            return carry
        lax.fori_loop(0, K, drain, 0)
    plsc.subcore_barrier()

    pltpu.sync_copy(acc.at[pl.ds(row0, ROWS_PER_TILE)],
                    agg_out.at[c, pl.ds(row0, ROWS_PER_TILE)])
    if with_deg:
        pltpu.sync_copy(dacc.at[pl.ds(row0, ROWS_PER_TILE)],
                        deg_out.at[c, pl.ds(row0, ROWS_PER_TILE)])


def _make_seg_kernel(with_deg):
    mesh = plsc.VectorSubcoreMesh(core_axis_name="c", subcore_axis_name="s")
    out_type = [jax.ShapeDtypeStruct((NC, NPAD, CB), jnp.float32)]
    scratch = [
        pltpu.VMEM((K, CHUNK), jnp.int32),
        pltpu.VMEM((K, CHUNK), jnp.int32),
    ]
    scratch += [pltpu.VMEM((CHUNK, CB), jnp.float32)] * NBUF
    scratch.append(pltpu.VMEM((ZR, CB), jnp.float32))
    if with_deg:
        out_type.append(jax.ShapeDtypeStruct((NC, NPAD, DEGW), jnp.float32))
        scratch += [
            pltpu.VMEM((CHUNK, DEGW), jnp.float32),
            pltpu.VMEM((ZR, DEGW), jnp.float32),
        ]
    scratch.append(pltpu.VMEM_SHARED((NPAD, CB), jnp.float32))
    if with_deg:
        scratch.append(pltpu.VMEM_SHARED((NPAD, DEGW), jnp.float32))
    scratch += [pltpu.SemaphoreType.DMA] * 3
    return pl.kernel(
        functools.partial(_seg_body, with_deg),
        out_type=out_type,
        mesh=mesh,
        scratch_types=scratch,
        compiler_params=pltpu.CompilerParams(use_tc_tiling_on_sc=False),
    )


_seg_deg = _make_seg_kernel(True)
_seg = _make_seg_kernel(False)


def _mm_body(x_ref, ws_ref, wn_ref, s_ref, y_ref):
    x = x_ref[...]
    s_ref[...] = jnp.dot(x, ws_ref[...], preferred_element_type=jnp.float32)
    y_ref[...] = jnp.dot(x, wn_ref[...], preferred_element_type=jnp.float32)


def _bn1_body(s1_ref, aggp_ref, degp_ref, ba_ref, ga_ref, be_ref, h_ref):
    agg = aggp_ref[0, :N, :] + aggp_ref[1, :N, :]
    deg = degp_ref[0, :N, 0:1] + degp_ref[1, :N, 0:1]
    dinv = 1.0 / jnp.clip(deg, 1.0, None)
    conv = s1_ref[...] + agg * dinv + ba_ref[...]
    mean = jnp.mean(conv, axis=0, keepdims=True)
    var = jnp.var(conv, axis=0, keepdims=True)
    bn = (conv - mean) * lax.rsqrt(var + 1e-5) * ga_ref[...] + be_ref[...]
    h_ref[...] = jnp.maximum(bn, 0.0)


def _bn2_body(h_ref, aggp_ref, degp_ref, data_ref, wbs_ref, wbn_ref,
              bb_ref, gb_ref, be_ref, out_ref):
    agg = aggp_ref[0, :N, :] + aggp_ref[1, :N, :]
    deg = degp_ref[0, :N, 0:1] + degp_ref[1, :N, 0:1]
    dinv = 1.0 / jnp.clip(deg, 1.0, None)
    conv = (jnp.dot(h_ref[...], wbs_ref[...],
                    preferred_element_type=jnp.float32)
            + jnp.dot(agg * dinv, wbn_ref[...],
                      preferred_element_type=jnp.float32)
            + bb_ref[...])
    mean = jnp.mean(conv, axis=0, keepdims=True)
    var = jnp.var(conv, axis=0, keepdims=True)
    bn = (conv - mean) * lax.rsqrt(var + 1e-5) * gb_ref[...] + be_ref[...]
    out_ref[...] = jnp.maximum(bn + data_ref[...], 0.0)


def kernel(data, edge_index, depth, Wa_self, Wa_neigh, ba, gamma_a, beta_a,
           Wb_self, Wb_neigh, bb, gamma_b, beta_b):
    del depth
    src = edge_index[0].astype(jnp.int32)
    dst = edge_index[1].astype(jnp.int32)
    pad = EP - E
    srcm = jnp.concatenate([src, jnp.zeros((pad,), jnp.int32)]).reshape(
        NW * K, CHUNK)
    dstm = jnp.concatenate([dst, jnp.full((pad,), JUNK_ROW, jnp.int32)]
                           ).reshape(NW * K, CHUNK)

    s1, y1 = pl.pallas_call(
        _mm_body,
        grid=(25,),
        in_specs=[
            pl.BlockSpec((400, C_IN), lambda i: (i, 0)),
            pl.BlockSpec((C_IN, CB), lambda i: (0, 0)),
            pl.BlockSpec((C_IN, CB), lambda i: (0, 0)),
        ],
        out_specs=[
            pl.BlockSpec((400, CB), lambda i: (i, 0)),
            pl.BlockSpec((400, CB), lambda i: (i, 0)),
        ],
        out_shape=[jax.ShapeDtypeStruct((N, CB), jnp.float32)] * 2,
    )(data, Wa_self, Wa_neigh)

    agg1_p, deg_p = _seg_deg(y1, srcm, dstm)

    h = pl.pallas_call(
        _bn1_body,
        out_shape=jax.ShapeDtypeStruct((N, CB), jnp.float32),
    )(s1, agg1_p, deg_p, ba.reshape(1, CB), gamma_a.reshape(1, CB),
      beta_a.reshape(1, CB))

    (agg2_p,) = _seg(h, srcm, dstm)

    out = pl.pallas_call(
        _bn2_body,
        out_shape=jax.ShapeDtypeStruct((N, C_IN), jnp.float32),
    )(h, agg2_p, deg_p, data, Wb_self, Wb_neigh, bb.reshape(1, C_IN),
      gamma_b.reshape(1, C_IN), beta_b.reshape(1, C_IN))
    return out


# speedup vs baseline: 11.8966x; 2.1382x over previous
import functools

import jax
import jax.numpy as jnp
from jax import lax
from jax.experimental import pallas as pl
from jax.experimental.pallas import tpu as pltpu
from jax.experimental.pallas import tpu_sc as plsc

N = 10000
E = 160000
C_IN = 256
CB = 64

NC = 2
NS = 16
NW = NC * NS
CHUNK = 128
K = 40
EP = NW * K * CHUNK
NPAD = 10112
ROWS_PER_TILE = NPAD // NS
JUNK_ROW = 10008
DEGW = 16
ZR = 64


NBUF = 4


def _seg_body(with_deg, table, srcm, dstm, *rest):
    if with_deg:
        (agg_out, deg_out, idx_s, idx_d, r0, r1, r2, r3, zbuf, ones, zbufd,
         acc, dacc, gsem, ssem, dsem) = rest
    else:
        (agg_out, idx_s, idx_d, r0, r1, r2, r3, zbuf,
         acc, gsem, ssem, dsem) = rest
    rows = (r0, r1, r2, r3)
    c = lax.axis_index("c")
    s = lax.axis_index("s")
    wid = c * NS + s

    zero16 = jnp.zeros((16,), jnp.float32)
    for r in range(ZR):
        for q in range(CB // 16):
            zbuf[r, pl.ds(q * 16, 16)] = zero16
    if with_deg:
        one16 = jnp.full((16,), 1.0, jnp.float32)
        for r in range(CHUNK):
            ones[r, pl.ds(0, 16)] = one16
        for r in range(ZR):
            zbufd[r, pl.ds(0, 16)] = zero16

    row0 = s * ROWS_PER_TILE
    nfull = ROWS_PER_TILE // ZR
    rem = ROWS_PER_TILE - nfull * ZR
    for k in range(nfull):
        pltpu.sync_copy(zbuf, acc.at[pl.ds(row0 + k * ZR, ZR)])
    pltpu.sync_copy(zbuf.at[pl.ds(0, rem)],
                    acc.at[pl.ds(row0 + nfull * ZR, rem)])
    if with_deg:
        for k in range(nfull):
            pltpu.sync_copy(zbufd, dacc.at[pl.ds(row0 + k * ZR, ZR)])
        pltpu.sync_copy(zbufd.at[pl.ds(0, rem)],
                        dacc.at[pl.ds(row0 + nfull * ZR, rem)])
    plsc.subcore_barrier()

    pltpu.sync_copy(srcm.at[pl.ds(wid * K, K)], idx_s)
    pltpu.sync_copy(dstm.at[pl.ds(wid * K, K)], idx_d)

    for b in range(NBUF):
        pltpu.async_copy(table.at[idx_s.at[b]], rows[b], gsem)

    def block(m, carry):
        j0 = m * NBUF
        for b in range(NBUF):
            j = j0 + b
            pltpu.make_async_copy(table.at[idx_s.at[j]], rows[b],
                                  gsem).wait()
            pltpu.async_copy(rows[b], acc.at[idx_d.at[j]], ssem, add=True)
            if with_deg:
                pltpu.async_copy(ones, dacc.at[idx_d.at[j]], dsem, add=True)
        for b in range(NBUF):
            j = j0 + b
            pltpu.make_async_copy(rows[b], acc.at[idx_d.at[j]], ssem).wait()

            @pl.when(j + NBUF < K)
            def _():
                pltpu.async_copy(table.at[idx_s.at[j + NBUF]], rows[b], gsem)
        return carry

    lax.fori_loop(0, K // NBUF, block, 0)
    if with_deg:
        def drain(j, carry):
            pltpu.make_async_copy(ones, dacc.at[idx_d.at[0]], dsem).wait()
            return carry
        lax.fori_loop(0, K, drain, 0)
    plsc.subcore_barrier()

    pltpu.sync_copy(acc.at[pl.ds(row0, ROWS_PER_TILE)],
                    agg_out.at[c, pl.ds(row0, ROWS_PER_TILE)])
    if with_deg:
        pltpu.sync_copy(dacc.at[pl.ds(row0, ROWS_PER_TILE)],
                        deg_out.at[c, pl.ds(row0, ROWS_PER_TILE)])


def _make_seg_kernel(with_deg):
    mesh = plsc.VectorSubcoreMesh(core_axis_name="c", subcore_axis_name="s")
    out_type = [jax.ShapeDtypeStruct((NC, NPAD, CB), jnp.float32)]
    scratch = [
        pltpu.VMEM((K, CHUNK), jnp.int32),
        pltpu.VMEM((K, CHUNK), jnp.int32),
    ]
    scratch += [pltpu.VMEM((CHUNK, CB), jnp.float32)] * NBUF
    scratch.append(pltpu.VMEM((ZR, CB), jnp.float32))
    if with_deg:
        out_type.append(jax.ShapeDtypeStruct((NC, NPAD, DEGW), jnp.float32))
        scratch += [
            pltpu.VMEM((CHUNK, DEGW), jnp.float32),
            pltpu.VMEM((ZR, DEGW), jnp.float32),
        ]
    scratch.append(pltpu.VMEM_SHARED((NPAD, CB), jnp.float32))
    if with_deg:
        scratch.append(pltpu.VMEM_SHARED((NPAD, DEGW), jnp.float32))
    scratch += [pltpu.SemaphoreType.DMA] * 3
    return pl.kernel(
        functools.partial(_seg_body, with_deg),
        out_type=out_type,
        mesh=mesh,
        scratch_types=scratch,
        compiler_params=pltpu.CompilerParams(use_tc_tiling_on_sc=False),
    )


_seg_deg = _make_seg_kernel(True)
_seg = _make_seg_kernel(False)


def _mm_body(x_ref, ws_ref, wn_ref, s_ref, y_ref):
    x = x_ref[...]
    s_ref[...] = jnp.dot(x, ws_ref[...], preferred_element_type=jnp.float32)
    y_ref[...] = jnp.dot(x, wn_ref[...], preferred_element_type=jnp.float32)


def _bn1_body(s1_ref, aggp_ref, degp_ref, ba_ref, ga_ref, be_ref, h_ref):
    agg = aggp_ref[0, :N, :] + aggp_ref[1, :N, :]
    deg = degp_ref[0, :N, 0:1] + degp_ref[1, :N, 0:1]
    dinv = 1.0 / jnp.clip(deg, 1.0, None)
    conv = s1_ref[...] + agg * dinv + ba_ref[...]
    mean = jnp.mean(conv, axis=0, keepdims=True)
    var = jnp.var(conv, axis=0, keepdims=True)
    bn = (conv - mean) * lax.rsqrt(var + 1e-5) * ga_ref[...] + be_ref[...]
    h_ref[...] = jnp.maximum(bn, 0.0)


def _bn2_body(h_ref, aggp_ref, degp_ref, data_ref, wbs_ref, wbn_ref,
              bb_ref, gb_ref, be_ref, out_ref):
    agg = aggp_ref[0, :N, :] + aggp_ref[1, :N, :]
    deg = degp_ref[0, :N, 0:1] + degp_ref[1, :N, 0:1]
    dinv = 1.0 / jnp.clip(deg, 1.0, None)
    conv = (jnp.dot(h_ref[...], wbs_ref[...],
                    preferred_element_type=jnp.float32)
            + jnp.dot(agg * dinv, wbn_ref[...],
                      preferred_element_type=jnp.float32)
            + bb_ref[...])
    mean = jnp.mean(conv, axis=0, keepdims=True)
    var = jnp.var(conv, axis=0, keepdims=True)
    bn = (conv - mean) * lax.rsqrt(var + 1e-5) * gb_ref[...] + be_ref[...]
    out_ref[...] = jnp.maximum(bn + data_ref[...], 0.0)


def kernel(data, edge_index, depth, Wa_self, Wa_neigh, ba, gamma_a, beta_a,
           Wb_self, Wb_neigh, bb, gamma_b, beta_b):
    del depth
    src = edge_index[0].astype(jnp.int32)
    dst = edge_index[1].astype(jnp.int32)
    pad = EP - E
    pad_src = (jnp.arange(pad, dtype=jnp.int32) * 61) % N
    pad_dst = N + jnp.arange(pad, dtype=jnp.int32) % (NPAD - N)
    srcm = jnp.concatenate([src, pad_src]).reshape(NW * K, CHUNK)
    dstm = jnp.concatenate([dst, pad_dst]).reshape(NW * K, CHUNK)

    s1, y1 = pl.pallas_call(
        _mm_body,
        grid=(25,),
        in_specs=[
            pl.BlockSpec((400, C_IN), lambda i: (i, 0)),
            pl.BlockSpec((C_IN, CB), lambda i: (0, 0)),
            pl.BlockSpec((C_IN, CB), lambda i: (0, 0)),
        ],
        out_specs=[
            pl.BlockSpec((400, CB), lambda i: (i, 0)),
            pl.BlockSpec((400, CB), lambda i: (i, 0)),
        ],
        out_shape=[jax.ShapeDtypeStruct((N, CB), jnp.float32)] * 2,
    )(data, Wa_self, Wa_neigh)

    agg1_p, deg_p = _seg_deg(y1, srcm, dstm)

    h = pl.pallas_call(
        _bn1_body,
        out_shape=jax.ShapeDtypeStruct((N, CB), jnp.float32),
    )(s1, agg1_p, deg_p, ba.reshape(1, CB), gamma_a.reshape(1, CB),
      beta_a.reshape(1, CB))

    (agg2_p,) = _seg(h, srcm, dstm)

    out = pl.pallas_call(
        _bn2_body,
        out_shape=jax.ShapeDtypeStruct((N, C_IN), jnp.float32),
    )(h, agg2_p, deg_p, data, Wb_self, Wb_neigh, bb.reshape(1, C_IN),
      gamma_b.reshape(1, C_IN), beta_b.reshape(1, C_IN))
    return out


# speedup vs baseline: 12.7054x; 1.0680x over previous
import functools

import jax
import jax.numpy as jnp
from jax import lax
from jax.experimental import pallas as pl
from jax.experimental.pallas import tpu as pltpu
from jax.experimental.pallas import tpu_sc as plsc

N = 10000
E = 160000
C_IN = 256
CB = 64

NC = 2
NS = 16
NW = NC * NS
CHUNK = 128
NCHUNKS = E // CHUNK
KBASE = NCHUNKS // NW
KEXTRA = NCHUNKS - KBASE * NW
K = KBASE + 1
NPAD = 10112
ROWS_PER_TILE = NPAD // NS
DEGW = 16
ZR = 64


NBUF = 4


def _seg_body(with_deg, table, eidx, *rest):
    if with_deg:
        (agg_out, deg_out, idx_s, idx_d, r0, r1, r2, r3, zbuf, ones, zbufd,
         acc, dacc, gsem, ssem, dsem) = rest
    else:
        (agg_out, idx_s, idx_d, r0, r1, r2, r3, zbuf,
         acc, gsem, ssem, dsem) = rest
    rows = (r0, r1, r2, r3)
    c = lax.axis_index("c")
    s = lax.axis_index("s")
    wid = c * NS + s

    zero16 = jnp.zeros((16,), jnp.float32)
    for r in range(ZR):
        for q in range(CB // 16):
            zbuf[r, pl.ds(q * 16, 16)] = zero16
    if with_deg:
        one16 = jnp.full((16,), 1.0, jnp.float32)
        for r in range(CHUNK):
            ones[r, pl.ds(0, 16)] = one16
        for r in range(ZR):
            zbufd[r, pl.ds(0, 16)] = zero16

    row0 = s * ROWS_PER_TILE
    nfull = ROWS_PER_TILE // ZR
    rem = ROWS_PER_TILE - nfull * ZR
    for k in range(nfull):
        pltpu.sync_copy(zbuf, acc.at[pl.ds(row0 + k * ZR, ZR)])
    pltpu.sync_copy(zbuf.at[pl.ds(0, rem)],
                    acc.at[pl.ds(row0 + nfull * ZR, rem)])
    if with_deg:
        for k in range(nfull):
            pltpu.sync_copy(zbufd, dacc.at[pl.ds(row0 + k * ZR, ZR)])
        pltpu.sync_copy(zbufd.at[pl.ds(0, rem)],
                        dacc.at[pl.ds(row0 + nfull * ZR, rem)])
    plsc.subcore_barrier()

    has_x = wid < KEXTRA
    base = KBASE * wid + jnp.minimum(wid, KEXTRA)
    pltpu.sync_copy(eidx.at[0, pl.ds(base, KBASE)], idx_s.at[pl.ds(0, KBASE)])
    pltpu.sync_copy(eidx.at[1, pl.ds(base, KBASE)], idx_d.at[pl.ds(0, KBASE)])

    @pl.when(has_x)
    def _():
        pltpu.sync_copy(eidx.at[0, pl.ds(base + KBASE, 1)],
                        idx_s.at[pl.ds(KBASE, 1)])
        pltpu.sync_copy(eidx.at[1, pl.ds(base + KBASE, 1)],
                        idx_d.at[pl.ds(KBASE, 1)])

    def guard(j, fn):
        if j < KBASE:
            fn()
        elif j == KBASE:
            pl.when(has_x)(fn)

    for b in range(NBUF):
        pltpu.async_copy(table.at[idx_s.at[b]], rows[b], gsem)

    for m in range(-(-K // NBUF)):
        for b in range(NBUF):
            j = m * NBUF + b
            if j >= K:
                continue

            def phase1(j=j, b=b):
                pltpu.make_async_copy(table.at[idx_s.at[j]], rows[b],
                                      gsem).wait()
                pltpu.async_copy(rows[b], acc.at[idx_d.at[j]], ssem,
                                 add=True)
                if with_deg:
                    pltpu.async_copy(ones, dacc.at[idx_d.at[j]], dsem,
                                     add=True)
            guard(j, phase1)
        for b in range(NBUF):
            j = m * NBUF + b
            if j >= K:
                continue

            def phase2(j=j, b=b):
                pltpu.make_async_copy(rows[b], acc.at[idx_d.at[j]],
                                      ssem).wait()
            guard(j, phase2)
            nj = j + NBUF

            def issue(nj=nj, b=b):
                pltpu.async_copy(table.at[idx_s.at[nj]], rows[b], gsem)
            if nj < K:
                guard(nj, issue)
    if with_deg:
        def drain(j, carry):
            pltpu.make_async_copy(ones, dacc.at[idx_d.at[0]], dsem).wait()
            return carry
        lax.fori_loop(0, KBASE, drain, 0)

        @pl.when(has_x)
        def _():
            pltpu.make_async_copy(ones, dacc.at[idx_d.at[0]], dsem).wait()
    plsc.subcore_barrier()

    pltpu.sync_copy(acc.at[pl.ds(row0, ROWS_PER_TILE)],
                    agg_out.at[c, pl.ds(row0, ROWS_PER_TILE)])
    if with_deg:
        pltpu.sync_copy(dacc.at[pl.ds(row0, ROWS_PER_TILE)],
                        deg_out.at[c, pl.ds(row0, ROWS_PER_TILE)])


def _make_seg_kernel(with_deg):
    mesh = plsc.VectorSubcoreMesh(core_axis_name="c", subcore_axis_name="s")
    out_type = [jax.ShapeDtypeStruct((NC, NPAD, CB), jnp.float32)]
    scratch = [
        pltpu.VMEM((K, CHUNK), jnp.int32),
        pltpu.VMEM((K, CHUNK), jnp.int32),
    ]
    scratch += [pltpu.VMEM((CHUNK, CB), jnp.float32)] * NBUF
    scratch.append(pltpu.VMEM((ZR, CB), jnp.float32))
    if with_deg:
        out_type.append(jax.ShapeDtypeStruct((NC, NPAD, DEGW), jnp.float32))
        scratch += [
            pltpu.VMEM((CHUNK, DEGW), jnp.float32),
            pltpu.VMEM((ZR, DEGW), jnp.float32),
        ]
    scratch.append(pltpu.VMEM_SHARED((NPAD, CB), jnp.float32))
    if with_deg:
        scratch.append(pltpu.VMEM_SHARED((NPAD, DEGW), jnp.float32))
    scratch += [pltpu.SemaphoreType.DMA] * 3
    return pl.kernel(
        functools.partial(_seg_body, with_deg),
        out_type=out_type,
        mesh=mesh,
        scratch_types=scratch,
        compiler_params=pltpu.CompilerParams(use_tc_tiling_on_sc=False),
    )


_seg_deg = _make_seg_kernel(True)
_seg = _make_seg_kernel(False)


def _mm_body(x_ref, ws_ref, wn_ref, s_ref, y_ref):
    x = x_ref[...]
    s_ref[...] = jnp.dot(x, ws_ref[...], preferred_element_type=jnp.float32)
    y_ref[...] = jnp.dot(x, wn_ref[...], preferred_element_type=jnp.float32)


def _bn1_body(s1_ref, aggp_ref, degp_ref, ba_ref, ga_ref, be_ref, h_ref):
    agg = aggp_ref[0, :N, :] + aggp_ref[1, :N, :]
    deg = degp_ref[0, :N, 0:1] + degp_ref[1, :N, 0:1]
    dinv = 1.0 / jnp.clip(deg, 1.0, None)
    conv = s1_ref[...] + agg * dinv + ba_ref[...]
    mean = jnp.mean(conv, axis=0, keepdims=True)
    var = jnp.var(conv, axis=0, keepdims=True)
    bn = (conv - mean) * lax.rsqrt(var + 1e-5) * ga_ref[...] + be_ref[...]
    h_ref[...] = jnp.maximum(bn, 0.0)


def _bn2_body(h_ref, aggp_ref, degp_ref, data_ref, wbs_ref, wbn_ref,
              bb_ref, gb_ref, be_ref, out_ref):
    agg = aggp_ref[0, :N, :] + aggp_ref[1, :N, :]
    deg = degp_ref[0, :N, 0:1] + degp_ref[1, :N, 0:1]
    dinv = 1.0 / jnp.clip(deg, 1.0, None)
    conv = (jnp.dot(h_ref[...], wbs_ref[...],
                    preferred_element_type=jnp.float32)
            + jnp.dot(agg * dinv, wbn_ref[...],
                      preferred_element_type=jnp.float32)
            + bb_ref[...])
    mean = jnp.mean(conv, axis=0, keepdims=True)
    var = jnp.var(conv, axis=0, keepdims=True)
    bn = (conv - mean) * lax.rsqrt(var + 1e-5) * gb_ref[...] + be_ref[...]
    out_ref[...] = jnp.maximum(bn + data_ref[...], 0.0)


def kernel(data, edge_index, depth, Wa_self, Wa_neigh, ba, gamma_a, beta_a,
           Wb_self, Wb_neigh, bb, gamma_b, beta_b):
    del depth
    eidx = edge_index.astype(jnp.int32).reshape(2, NCHUNKS, CHUNK)

    s1, y1 = pl.pallas_call(
        _mm_body,
        grid=(10,),
        in_specs=[
            pl.BlockSpec((1000, C_IN), lambda i: (i, 0)),
            pl.BlockSpec((C_IN, CB), lambda i: (0, 0)),
            pl.BlockSpec((C_IN, CB), lambda i: (0, 0)),
        ],
        out_specs=[
            pl.BlockSpec((1000, CB), lambda i: (i, 0)),
            pl.BlockSpec((1000, CB), lambda i: (i, 0)),
        ],
        out_shape=[jax.ShapeDtypeStruct((N, CB), jnp.float32)] * 2,
    )(data, Wa_self, Wa_neigh)

    agg1_p, deg_p = _seg_deg(y1, eidx)

    h = pl.pallas_call(
        _bn1_body,
        out_shape=jax.ShapeDtypeStruct((N, CB), jnp.float32),
    )(s1, agg1_p, deg_p, ba.reshape(1, CB), gamma_a.reshape(1, CB),
      beta_a.reshape(1, CB))

    (agg2_p,) = _seg(h, eidx)

    out = pl.pallas_call(
        _bn2_body,
        out_shape=jax.ShapeDtypeStruct((N, C_IN), jnp.float32),
    )(h, agg2_p, deg_p, data, Wb_self, Wb_neigh, bb.reshape(1, C_IN),
      gamma_b.reshape(1, C_IN), beta_b.reshape(1, C_IN))
    return out
